# baseline (speedup 1.0000x reference)
import functools

import jax
import jax.numpy as jnp
from jax import lax
from jax.experimental import pallas as pl
from jax.experimental.pallas import tpu as pltpu
from jax.experimental.pallas import tpu_sc as plsc

N = 10000
E = 160000
F_IN = 128
F_OUT = 64

NC = 2
NS = 16
NW = NC * NS

F_SC = 64
B_EDGE = 128
CH0 = 52
CH1 = 28
CH_MAX = max(CH0, CH1)
E_PAD = NS * B_EDGE * (CH0 + CH1)
N_PAD = 10112
ROWS_PER_TILE = N_PAD // NS


def _feat_matmul(h, Wp):
    def body(h_ref, w_ref, o_ref):
        o_ref[...] = jnp.dot(h_ref[...], w_ref[...],
                             preferred_element_type=jnp.float32)

    return pl.pallas_call(
        body,
        out_shape=jax.ShapeDtypeStruct((N, F_SC), jnp.float32),
    )(h, Wp)


UNROLL = 4


def _make_sc_agg():
    mesh = plsc.VectorSubcoreMesh(core_axis_name="c", subcore_axis_name="s")

    @functools.partial(
        pl.kernel,
        mesh=mesh,
        out_type=jax.ShapeDtypeStruct((NC, N_PAD, F_SC), jnp.float32),
        scratch_types=[
            pltpu.VMEM((CH_MAX, B_EDGE), jnp.int32),
            pltpu.VMEM((CH_MAX, B_EDGE), jnp.int32),
        ] + [pltpu.VMEM((B_EDGE, F_SC), jnp.float32)] * UNROLL
          + [pltpu.VMEM_SHARED((N_PAD, F_SC), jnp.float32)]
          + [pltpu.SemaphoreType.DMA] * (2 * UNROLL),
        compiler_params=pltpu.CompilerParams(use_tc_tiling_on_sc=False),
    )
    def sc_agg(feat_hbm, src_hbm, dst_hbm, zeros_hbm, out_hbm,
               src_v, dst_v, *bufs_and_sems):
        rows = bufs_and_sems[:UNROLL]
        agg_sh = bufs_and_sems[UNROLL]
        sem_g = bufs_and_sems[UNROLL + 1:2 * UNROLL + 1]
        sem_s = bufs_and_sems[2 * UNROLL + 1:]
        cid = lax.axis_index("c")
        sid = lax.axis_index("s")
        row0 = sid * ROWS_PER_TILE

        pltpu.sync_copy(zeros_hbm.at[pl.ds(row0, ROWS_PER_TILE)],
                        agg_sh.at[pl.ds(row0, ROWS_PER_TILE)])
        pltpu.sync_copy(src_hbm.at[cid, sid], src_v)
        pltpu.sync_copy(dst_hbm.at[cid, sid], dst_v)
        plsc.subcore_barrier()

        def body(t, carry):
            c0 = t * UNROLL
            gathers = [
                pltpu.async_copy(feat_hbm.at[src_v.at[c0 + u]], rows[u],
                                 sem_g[u])
                for u in range(UNROLL)
            ]
            scats = []
            for u in range(UNROLL):
                gathers[u].wait()
                scats.append(
                    pltpu.async_copy(rows[u], agg_sh.at[dst_v.at[c0 + u]],
                                     sem_s[u], add=True))
            for s in scats:
                s.wait()
            return carry

        n_outer = jnp.where(cid == 0, CH0 // UNROLL, CH1 // UNROLL)
        lax.fori_loop(0, n_outer, body, 0)
        plsc.subcore_barrier()

        pltpu.sync_copy(agg_sh.at[pl.ds(row0, ROWS_PER_TILE)],
                        out_hbm.at[cid, pl.ds(row0, ROWS_PER_TILE)])

    return sc_agg


_sc_agg = _make_sc_agg()


def _x_build(agg, b2d):
    def body(a_ref, b_ref, o_ref):
        s = a_ref[0] + a_ref[1]
        o_ref[...] = jnp.maximum(s[:, :F_OUT] + b_ref[...], 0.0)

    return pl.pallas_call(
        body,
        out_shape=jax.ShapeDtypeStruct((N_PAD, F_OUT), jnp.float32),
    )(agg, b2d)


BI = 512
BJ = 10240


def _xxt(x):
    gi = pl.cdiv(N, BI)
    gj = pl.cdiv(N, BJ)

    def body(xi_ref, xj_ref, o_ref):
        o_ref[...] = lax.dot_general(
            xi_ref[...], xj_ref[...],
            (((1,), (1,)), ((), ())),
            preferred_element_type=jnp.float32)

    return pl.pallas_call(
        body,
        grid=(gj, gi),
        in_specs=[
            pl.BlockSpec((BI, F_OUT), lambda j, i: (i, 0)),
            pl.BlockSpec((BJ, F_OUT), lambda j, i: (j, 0)),
        ],
        out_specs=pl.BlockSpec((BI, BJ), lambda j, i: (i, j)),
        out_shape=jax.ShapeDtypeStruct((N, N), jnp.float32),
        compiler_params=pltpu.CompilerParams(
            dimension_semantics=("parallel", "parallel")),
    )(x, x)


def kernel(h, edge_index, W, b):
    src = edge_index[0].astype(jnp.int32)
    dst = edge_index[1].astype(jnp.int32)
    pad = E_PAD - E
    e0 = NS * CH0 * B_EDGE
    src_f = jnp.concatenate([src, jnp.zeros((pad,), jnp.int32)])
    dst_f = jnp.concatenate([dst, jnp.full((pad,), N, jnp.int32)])

    def _split(flat, fill):
        c0 = flat[:e0].reshape(NS, CH0, B_EDGE)
        c0 = jnp.pad(c0, ((0, 0), (0, CH_MAX - CH0), (0, 0)),
                     constant_values=fill)
        c1 = flat[e0:].reshape(NS, CH1, B_EDGE)
        c1 = jnp.pad(c1, ((0, 0), (0, CH_MAX - CH1), (0, 0)),
                     constant_values=fill)
        return jnp.stack([c0, c1])

    src_p = _split(src_f, 0)
    dst_p = _split(dst_f, N)

    feat = _feat_matmul(h, W)
    zeros = jnp.zeros((N_PAD, F_SC), jnp.float32)
    agg = _sc_agg(feat, src_p, dst_p, zeros)
    x = _x_build(agg, b.reshape(1, F_OUT))
    return _xxt(x)


# speedup vs baseline: 1.1354x; 1.1354x over previous
import functools

import jax
import jax.numpy as jnp
from jax import lax
from jax.experimental import pallas as pl
from jax.experimental.pallas import tpu as pltpu
from jax.experimental.pallas import tpu_sc as plsc

N = 10000
E = 160000
F_IN = 128
F_OUT = 64

NC = 2
NS = 16
NW = NC * NS

F_SC = 64
B_EDGE = 128
CH0 = 60
CH1 = 20
CH_MAX = max(CH0, CH1)
E_PAD = NS * B_EDGE * (CH0 + CH1)
N_PAD = 10112
ROWS_PER_TILE = N_PAD // NS


def _feat_matmul(h, Wp):
    def body(h_ref, w_ref, o_ref):
        o_ref[...] = jnp.dot(h_ref[...], w_ref[...],
                             preferred_element_type=jnp.float32)

    return pl.pallas_call(
        body,
        out_shape=jax.ShapeDtypeStruct((N, F_SC), jnp.float32),
    )(h, Wp)


UNROLL = 4


def _make_sc_agg():
    mesh = plsc.VectorSubcoreMesh(core_axis_name="c", subcore_axis_name="s")

    @functools.partial(
        pl.kernel,
        mesh=mesh,
        out_type=jax.ShapeDtypeStruct((NC, N_PAD, F_SC), jnp.float32),
        scratch_types=[
            pltpu.VMEM((CH_MAX, B_EDGE), jnp.int32),
            pltpu.VMEM((CH_MAX, B_EDGE), jnp.int32),
        ] + [pltpu.VMEM((B_EDGE, F_SC), jnp.float32)] * UNROLL
          + [pltpu.VMEM_SHARED((N_PAD, F_SC), jnp.float32)]
          + [pltpu.SemaphoreType.DMA] * (2 * UNROLL),
        compiler_params=pltpu.CompilerParams(use_tc_tiling_on_sc=False),
    )
    def sc_agg(feat_hbm, src_hbm, dst_hbm, zeros_hbm, out_hbm,
               src_v, dst_v, *bufs_and_sems):
        rows = bufs_and_sems[:UNROLL]
        agg_sh = bufs_and_sems[UNROLL]
        sem_g = bufs_and_sems[UNROLL + 1:2 * UNROLL + 1]
        sem_s = bufs_and_sems[2 * UNROLL + 1:]
        cid = lax.axis_index("c")
        sid = lax.axis_index("s")
        row0 = sid * ROWS_PER_TILE

        pltpu.sync_copy(zeros_hbm.at[pl.ds(row0, ROWS_PER_TILE)],
                        agg_sh.at[pl.ds(row0, ROWS_PER_TILE)])
        pltpu.sync_copy(src_hbm.at[cid, sid], src_v)
        pltpu.sync_copy(dst_hbm.at[cid, sid], dst_v)
        plsc.subcore_barrier()

        def body(t, carry):
            c0 = t * UNROLL
            gathers = [
                pltpu.async_copy(feat_hbm.at[src_v.at[c0 + u]], rows[u],
                                 sem_g[u])
                for u in range(UNROLL)
            ]
            scats = []
            for u in range(UNROLL):
                gathers[u].wait()
                scats.append(
                    pltpu.async_copy(rows[u], agg_sh.at[dst_v.at[c0 + u]],
                                     sem_s[u], add=True))
            for s in scats:
                s.wait()
            return carry

        n_outer = jnp.where(cid == 0, CH0 // UNROLL, CH1 // UNROLL)
        lax.fori_loop(0, n_outer, body, 0)
        plsc.subcore_barrier()

        pltpu.sync_copy(agg_sh.at[pl.ds(row0, ROWS_PER_TILE)],
                        out_hbm.at[cid, pl.ds(row0, ROWS_PER_TILE)])

    return sc_agg


_sc_agg = _make_sc_agg()


def _x_build(agg, b2d):
    def body(a_ref, b_ref, o_ref):
        s = a_ref[0] + a_ref[1]
        o_ref[...] = jnp.maximum(s[:, :F_OUT] + b_ref[...], 0.0)

    return pl.pallas_call(
        body,
        out_shape=jax.ShapeDtypeStruct((N_PAD, F_OUT), jnp.float32),
    )(agg, b2d)


BI = 512
BJ = 10240


def _xxt(x):
    gi = pl.cdiv(N, BI)
    gj = pl.cdiv(N, BJ)

    def body(xi_ref, xj_ref, o_ref):
        o_ref[...] = lax.dot_general(
            xi_ref[...], xj_ref[...],
            (((1,), (1,)), ((), ())),
            preferred_element_type=jnp.float32)

    return pl.pallas_call(
        body,
        grid=(gj, gi),
        in_specs=[
            pl.BlockSpec((BI, F_OUT), lambda j, i: (i, 0)),
            pl.BlockSpec((BJ, F_OUT), lambda j, i: (j, 0)),
        ],
        out_specs=pl.BlockSpec((BI, BJ), lambda j, i: (i, j)),
        out_shape=jax.ShapeDtypeStruct((N, N), jnp.float32),
        compiler_params=pltpu.CompilerParams(
            dimension_semantics=("parallel", "parallel")),
    )(x, x)


def kernel(h, edge_index, W, b):
    src = edge_index[0].astype(jnp.int32)
    dst = edge_index[1].astype(jnp.int32)
    pad = E_PAD - E
    e0 = NS * CH0 * B_EDGE
    src_f = jnp.concatenate([src, jnp.zeros((pad,), jnp.int32)])
    dst_f = jnp.concatenate([dst, jnp.full((pad,), N, jnp.int32)])

    def _split(flat, fill):
        c0 = flat[:e0].reshape(NS, CH0, B_EDGE)
        c0 = jnp.pad(c0, ((0, 0), (0, CH_MAX - CH0), (0, 0)),
                     constant_values=fill)
        c1 = flat[e0:].reshape(NS, CH1, B_EDGE)
        c1 = jnp.pad(c1, ((0, 0), (0, CH_MAX - CH1), (0, 0)),
                     constant_values=fill)
        return jnp.stack([c0, c1])

    src_p = _split(src_f, 0)
    dst_p = _split(dst_f, N)

    feat = _feat_matmul(h, W)
    zeros = jnp.zeros((N_PAD, F_SC), jnp.float32)
    agg = _sc_agg(feat, src_p, dst_p, zeros)
    x = _x_build(agg, b.reshape(1, F_OUT))
    return _xxt(x)


# speedup vs baseline: 1.1471x; 1.0103x over previous
import functools

import jax
import jax.numpy as jnp
from jax import lax
from jax.experimental import pallas as pl
from jax.experimental.pallas import tpu as pltpu
from jax.experimental.pallas import tpu_sc as plsc

N = 10000
E = 160000
F_IN = 128
F_OUT = 64

NC = 2
NS = 16
NW = NC * NS

F_SC = 64
B_EDGE = 128
CH0 = 60
CH1 = 20
CH_MAX = max(CH0, CH1)
E_PAD = NS * B_EDGE * (CH0 + CH1)
N_PAD = 10112
ROWS_PER_TILE = N_PAD // NS


def _feat_matmul(h, Wp):
    def body(h_ref, w_ref, o_ref):
        o_ref[...] = jnp.dot(h_ref[...], w_ref[...],
                             preferred_element_type=jnp.float32)

    return pl.pallas_call(
        body,
        out_shape=jax.ShapeDtypeStruct((N, F_SC), jnp.float32),
    )(h, Wp)


UNROLL = 4


def _make_sc_agg():
    mesh = plsc.VectorSubcoreMesh(core_axis_name="c", subcore_axis_name="s")

    @functools.partial(
        pl.kernel,
        mesh=mesh,
        out_type=jax.ShapeDtypeStruct((NC, N_PAD, F_SC), jnp.float32),
        scratch_types=[
            pltpu.VMEM((CH_MAX, B_EDGE), jnp.int32),
            pltpu.VMEM((CH_MAX, B_EDGE), jnp.int32),
        ] + [pltpu.VMEM((B_EDGE, F_SC), jnp.float32)] * UNROLL
          + [pltpu.VMEM_SHARED((N_PAD, F_SC), jnp.float32)]
          + [pltpu.SemaphoreType.DMA] * (2 * UNROLL),
        compiler_params=pltpu.CompilerParams(use_tc_tiling_on_sc=False),
    )
    def sc_agg(feat_hbm, src_hbm, dst_hbm, zeros_hbm, out_hbm,
               src_v, dst_v, *bufs_and_sems):
        rows = bufs_and_sems[:UNROLL]
        agg_sh = bufs_and_sems[UNROLL]
        sem_g = bufs_and_sems[UNROLL + 1:2 * UNROLL + 1]
        sem_s = bufs_and_sems[2 * UNROLL + 1:]
        cid = lax.axis_index("c")
        sid = lax.axis_index("s")
        row0 = sid * ROWS_PER_TILE

        pltpu.sync_copy(zeros_hbm.at[pl.ds(row0, ROWS_PER_TILE)],
                        agg_sh.at[pl.ds(row0, ROWS_PER_TILE)])
        pltpu.sync_copy(src_hbm.at[cid, sid], src_v)
        pltpu.sync_copy(dst_hbm.at[cid, sid], dst_v)
        plsc.subcore_barrier()

        def body(t, carry):
            c0 = t * UNROLL
            gathers = [
                pltpu.async_copy(feat_hbm.at[src_v.at[c0 + u]], rows[u],
                                 sem_g[u])
                for u in range(UNROLL)
            ]
            scats = []
            for u in range(UNROLL):
                gathers[u].wait()
                scats.append(
                    pltpu.async_copy(rows[u], agg_sh.at[dst_v.at[c0 + u]],
                                     sem_s[u], add=True))
            for s in scats:
                s.wait()
            return carry

        n_outer = jnp.where(cid == 0, CH0 // UNROLL, CH1 // UNROLL)
        lax.fori_loop(0, n_outer, body, 0)
        plsc.subcore_barrier()

        pltpu.sync_copy(agg_sh.at[pl.ds(row0, ROWS_PER_TILE)],
                        out_hbm.at[cid, pl.ds(row0, ROWS_PER_TILE)])

    return sc_agg


_sc_agg = _make_sc_agg()


def _x_build(agg, b2d):
    def body(a_ref, b_ref, o_ref):
        s = a_ref[0] + a_ref[1]
        o_ref[...] = jnp.maximum(s[:, :F_OUT] + b_ref[...],
                                 0.0).astype(jnp.bfloat16)

    return pl.pallas_call(
        body,
        out_shape=jax.ShapeDtypeStruct((N_PAD, F_OUT), jnp.bfloat16),
    )(agg, b2d)


BI = 512
BJ = 10240


def _xxt(x):
    gi = pl.cdiv(N, BI)
    gj = pl.cdiv(N, BJ)

    def body(xi_ref, xj_ref, o_ref):
        o_ref[...] = lax.dot_general(
            xi_ref[...], xj_ref[...],
            (((1,), (1,)), ((), ())),
            preferred_element_type=jnp.float32)

    return pl.pallas_call(
        body,
        grid=(gj, gi),
        in_specs=[
            pl.BlockSpec((BI, F_OUT), lambda j, i: (i, 0)),
            pl.BlockSpec((BJ, F_OUT), lambda j, i: (j, 0)),
        ],
        out_specs=pl.BlockSpec((BI, BJ), lambda j, i: (i, j)),
        out_shape=jax.ShapeDtypeStruct((N, N), jnp.float32),
        compiler_params=pltpu.CompilerParams(
            dimension_semantics=("parallel", "parallel")),
    )(x, x)


def kernel(h, edge_index, W, b):
    src = edge_index[0].astype(jnp.int32)
    dst = edge_index[1].astype(jnp.int32)
    pad = E_PAD - E
    e0 = NS * CH0 * B_EDGE
    src_f = jnp.concatenate([src, jnp.zeros((pad,), jnp.int32)])
    dst_f = jnp.concatenate([dst, jnp.full((pad,), N, jnp.int32)])

    def _split(flat, fill):
        c0 = flat[:e0].reshape(NS, CH0, B_EDGE)
        c0 = jnp.pad(c0, ((0, 0), (0, CH_MAX - CH0), (0, 0)),
                     constant_values=fill)
        c1 = flat[e0:].reshape(NS, CH1, B_EDGE)
        c1 = jnp.pad(c1, ((0, 0), (0, CH_MAX - CH1), (0, 0)),
                     constant_values=fill)
        return jnp.stack([c0, c1])

    src_p = _split(src_f, 0)
    dst_p = _split(dst_f, N)

    feat = _feat_matmul(h, W)
    zeros = jnp.zeros((N_PAD, F_SC), jnp.float32)
    agg = _sc_agg(feat, src_p, dst_p, zeros)
    x = _x_build(agg, b.reshape(1, F_OUT))
    return _xxt(x)
